# baseline (speedup 1.0000x reference)
import jax
import jax.numpy as jnp
from jax.experimental import pallas as pl
from jax.experimental.pallas import tpu as pltpu

S = 1024
D = 1024
H = 16
DH = 64
FF = 4096
DEPTH = 4
SCALE = 1.0 / 8.0


def _ln(x, eps=1e-5):
    mu = jnp.mean(x, axis=-1, keepdims=True)
    var = jnp.mean((x - mu) ** 2, axis=-1, keepdims=True)
    return (x - mu) / jnp.sqrt(var + eps)


def _split(a):
    ah = a.astype(jnp.bfloat16)
    al = (a - ah.astype(jnp.float32)).astype(jnp.bfloat16)
    return ah, al


def _dot(x, y):
    return jnp.dot(x, y, preferred_element_type=jnp.float32)


def _mm3(a, b):
    ah, al = _split(a)
    bh, bl = _split(b)
    return (_dot(ah, bl) + _dot(al, bh)) + _dot(ah, bh)


def _mm3_pre(ah, al, b):
    bh, bl = _split(b)
    return (_dot(ah, bl) + _dot(al, bh)) + _dot(ah, bh)


def _mm3_t(a, b):
    ah, al = _split(a)
    bh, bl = _split(b)
    dn = (((1,), (1,)), ((), ()))
    d = lambda x, y: jax.lax.dot_general(x, y, dn,
                                         preferred_element_type=jnp.float32)
    return (d(ah, bl) + d(al, bh)) + d(ah, bh)


def _embed_body(x_ref, we_ref, be_ref, pos_ref, out_ref):
    out_ref[...] = (
        jnp.dot(x_ref[...], we_ref[...], preferred_element_type=jnp.float32, precision=jax.lax.Precision.HIGHEST)
        + be_ref[...]
        + pos_ref[...]
    )


def _attn_body(h_ref, wqk_ref, wv_ref, wo_ref, out_ref, xh_s, xl_s, bias_s):
    hd = pl.program_id(0)

    @pl.when(hd == 0)
    def _():
        xn = _ln(h_ref[...])
        xh = xn.astype(jnp.bfloat16)
        xh_s[...] = xh
        xl_s[...] = (xn - xh.astype(jnp.float32)).astype(jnp.bfloat16)
        row = jax.lax.broadcasted_iota(jnp.int32, (S, S), 0)
        col = jax.lax.broadcasted_iota(jnp.int32, (S, S), 1)
        bias_s[...] = jnp.where(col > row, -1e9,
                                jnp.where(col == row, -1e5, 0.0))
        out_ref[...] = h_ref[...]

    xh = xh_s[...]
    xl = xl_s[...]
    bias = bias_s[...]
    q2 = _mm3_pre(xh, xl, wqk_ref[...])
    v2 = _mm3_pre(xh, xl, wv_ref[...])
    ohs = []
    for i in range(2):
        q = q2[:, i * DH:(i + 1) * DH]
        nrm = jnp.sqrt(jnp.sum(q * q, axis=1, keepdims=True)) + 1e-9
        kn = q / nrm
        dots = _mm3_t(q * SCALE, kn) + bias
        m = jnp.max(dots, axis=1, keepdims=True)
        p = jnp.exp(dots - m)
        s = jnp.sum(p, axis=1, keepdims=True)
        oh = _mm3(p, v2[:, i * DH:(i + 1) * DH]) / s
        ohs.append(oh)
    o2 = jnp.concatenate(ohs, axis=1)
    out_ref[...] += _mm3(o2, wo_ref[...])


def _ffn_body(h_ref, w1_ref, b1_ref, w2_ref, b2_ref, out_ref, xh_s, xl_s):
    c = pl.program_id(0)

    @pl.when(c == 0)
    def _():
        xn = _ln(h_ref[...])
        xh = xn.astype(jnp.bfloat16)
        xh_s[...] = xh
        xl_s[...] = (xn - xh.astype(jnp.float32)).astype(jnp.bfloat16)
        out_ref[...] = h_ref[...] + b2_ref[...]

    g = jax.nn.gelu(_mm3_pre(xh_s[...], xl_s[...], w1_ref[...]) + b1_ref[...])
    out_ref[...] += _mm3(g, w2_ref[...])


def _head_body(h_ref, wf1_ref, bf1_ref, wf2t_ref, bf2_ref, out_ref):
    last = h_ref[S - 1:S, :]
    ln1 = _ln(last)
    z = jnp.dot(ln1, wf1_ref[...], preferred_element_type=jnp.float32, precision=jax.lax.Precision.HIGHEST) + bf1_ref[...]
    z = jax.nn.relu(_ln(z))
    out_ref[...] = jnp.sum(z * wf2t_ref[...], axis=1, keepdims=True) + bf2_ref[...]


def kernel(x, W_emb, b_emb, pos, Wqk, Wv, Wo, W1, b1, W2, b2,
           Wf1, bf1, Wf2, bf2, R):
    f32 = jnp.float32
    h = pl.pallas_call(
        _embed_body,
        out_shape=jax.ShapeDtypeStruct((S, D), f32),
    )(x[0], W_emb, b_emb.reshape(1, D), pos[0])

    attn = pl.pallas_call(
        _attn_body,
        grid=(H // 2,),
        in_specs=[
            pl.BlockSpec((S, D), lambda hd: (0, 0)),
            pl.BlockSpec((D, 2 * DH), lambda hd: (0, hd)),
            pl.BlockSpec((D, 2 * DH), lambda hd: (0, hd)),
            pl.BlockSpec((2 * DH, D), lambda hd: (hd, 0)),
        ],
        out_specs=pl.BlockSpec((S, D), lambda hd: (0, 0)),
        out_shape=jax.ShapeDtypeStruct((S, D), f32),
        scratch_shapes=[pltpu.VMEM((S, D), jnp.bfloat16),
                        pltpu.VMEM((S, D), jnp.bfloat16),
                        pltpu.VMEM((S, S), f32)],
    )

    ffc = FF // 4
    ffn = pl.pallas_call(
        _ffn_body,
        grid=(4,),
        in_specs=[
            pl.BlockSpec((S, D), lambda c: (0, 0)),
            pl.BlockSpec((D, ffc), lambda c: (0, c)),
            pl.BlockSpec((1, ffc), lambda c: (0, c)),
            pl.BlockSpec((ffc, D), lambda c: (c, 0)),
            pl.BlockSpec((1, D), lambda c: (0, 0)),
        ],
        out_specs=pl.BlockSpec((S, D), lambda c: (0, 0)),
        out_shape=jax.ShapeDtypeStruct((S, D), f32),
        scratch_shapes=[pltpu.VMEM((S, D), jnp.bfloat16),
                        pltpu.VMEM((S, D), jnp.bfloat16)],
    )

    for l in range(DEPTH):
        h = attn(h, Wqk[l], Wv[l], Wo[l])
        h = ffn(h, W1[l], b1[l].reshape(1, FF), W2[l], b2[l].reshape(1, D))

    out = pl.pallas_call(
        _head_body,
        out_shape=jax.ShapeDtypeStruct((1, 1), f32),
    )(h, Wf1, bf1.reshape(1, D // 2), Wf2.reshape(1, D // 2), bf2.reshape(1, 1))
    return out


# speedup vs baseline: 1.8902x; 1.8902x over previous
import jax
import jax.numpy as jnp
from jax.experimental import pallas as pl
from jax.experimental.pallas import tpu as pltpu

S = 1024
D = 1024
H = 16
DH = 64
FF = 4096
DEPTH = 4
SCALE = 1.0 / 8.0


def _ln(x, eps=1e-5):
    mu = jnp.mean(x, axis=-1, keepdims=True)
    var = jnp.mean((x - mu) ** 2, axis=-1, keepdims=True)
    return (x - mu) * jax.lax.rsqrt(var + eps)


def _bf(a):
    return a.astype(jnp.bfloat16)


def _mmb(a, b):
    return jnp.dot(_bf(a), _bf(b), preferred_element_type=jnp.float32)


def _mmb_t(a, b):
    dn = (((1,), (1,)), ((), ()))
    return jax.lax.dot_general(_bf(a), _bf(b), dn,
                               preferred_element_type=jnp.float32)


def _embed_body(x_ref, we_ref, be_ref, pos_ref, out_ref):
    out_ref[...] = (
        jnp.dot(x_ref[...], we_ref[...], preferred_element_type=jnp.float32,
                precision=jax.lax.Precision.HIGHEST)
        + be_ref[...]
        + pos_ref[...]
    )


def _attn_body(h_ref, wqk_ref, wv_ref, wo_ref, out_ref, xh_s, bias_s):
    hd = pl.program_id(0)

    @pl.when(hd == 0)
    def _():
        xn = _ln(h_ref[...])
        xh_s[...] = xn.astype(jnp.bfloat16)
        row = jax.lax.broadcasted_iota(jnp.int32, (S, S), 0)
        col = jax.lax.broadcasted_iota(jnp.int32, (S, S), 1)
        bias_s[...] = jnp.where(col > row, -1e9,
                                jnp.where(col == row, -1e5, 0.0))
        out_ref[...] = h_ref[...]

    xh = xh_s[...]
    bias = bias_s[...]
    q2 = jnp.dot(xh, _bf(wqk_ref[...]), preferred_element_type=jnp.float32)
    v2 = jnp.dot(xh, _bf(wv_ref[...]), preferred_element_type=jnp.float32)
    ohs = []
    for i in range(2):
        q = q2[:, i * DH:(i + 1) * DH]
        nrm = jnp.sqrt(jnp.sum(q * q, axis=1, keepdims=True)) + 1e-9
        kn = q / nrm
        dots = _mmb_t(q, kn) * SCALE + bias
        m = jnp.max(dots, axis=1, keepdims=True)
        p = jnp.exp(dots - m)
        s = jnp.sum(p, axis=1, keepdims=True)
        slog = m + jnp.log(s)
        probs = jnp.exp(dots - slog)
        oh = _mmb(probs, v2[:, i * DH:(i + 1) * DH])
        ohs.append(oh)
    o2 = jnp.concatenate(ohs, axis=1)
    out_ref[...] += _mmb(o2, wo_ref[...])


def _ffn_body(h_ref, w1_ref, b1_ref, w2_ref, b2_ref, out_ref, xh_s):
    c = pl.program_id(0)

    @pl.when(c == 0)
    def _():
        xn = _ln(h_ref[...])
        xh_s[...] = xn.astype(jnp.bfloat16)
        out_ref[...] = h_ref[...] + b2_ref[...]

    g = jax.nn.gelu(
        jnp.dot(xh_s[...], _bf(w1_ref[...]), preferred_element_type=jnp.float32)
        + b1_ref[...])
    out_ref[...] += _mmb(g, w2_ref[...])


def _head_body(h_ref, wf1_ref, bf1_ref, wf2t_ref, bf2_ref, out_ref):
    last = h_ref[S - 1:S, :]
    ln1 = _ln(last)
    z = _mmb(ln1, wf1_ref[...]) + bf1_ref[...]
    z = jax.nn.relu(_ln(z))
    out_ref[...] = jnp.sum(z * wf2t_ref[...], axis=1, keepdims=True) + bf2_ref[...]


def kernel(x, W_emb, b_emb, pos, Wqk, Wv, Wo, W1, b1, W2, b2,
           Wf1, bf1, Wf2, bf2, R):
    f32 = jnp.float32
    h = pl.pallas_call(
        _embed_body,
        out_shape=jax.ShapeDtypeStruct((S, D), f32),
    )(x[0], W_emb, b_emb.reshape(1, D), pos[0])

    attn = pl.pallas_call(
        _attn_body,
        grid=(H // 2,),
        in_specs=[
            pl.BlockSpec((S, D), lambda hd: (0, 0)),
            pl.BlockSpec((D, 2 * DH), lambda hd: (0, hd)),
            pl.BlockSpec((D, 2 * DH), lambda hd: (0, hd)),
            pl.BlockSpec((2 * DH, D), lambda hd: (hd, 0)),
        ],
        out_specs=pl.BlockSpec((S, D), lambda hd: (0, 0)),
        out_shape=jax.ShapeDtypeStruct((S, D), f32),
        scratch_shapes=[pltpu.VMEM((S, D), jnp.bfloat16),
                        pltpu.VMEM((S, S), f32)],
    )

    ffc = FF // 4
    ffn = pl.pallas_call(
        _ffn_body,
        grid=(4,),
        in_specs=[
            pl.BlockSpec((S, D), lambda c: (0, 0)),
            pl.BlockSpec((D, ffc), lambda c: (0, c)),
            pl.BlockSpec((1, ffc), lambda c: (0, c)),
            pl.BlockSpec((ffc, D), lambda c: (c, 0)),
            pl.BlockSpec((1, D), lambda c: (0, 0)),
        ],
        out_specs=pl.BlockSpec((S, D), lambda c: (0, 0)),
        out_shape=jax.ShapeDtypeStruct((S, D), f32),
        scratch_shapes=[pltpu.VMEM((S, D), jnp.bfloat16)],
    )

    for l in range(DEPTH):
        h = attn(h, Wqk[l], Wv[l], Wo[l])
        h = ffn(h, W1[l], b1[l].reshape(1, FF), W2[l], b2[l].reshape(1, D))

    out = pl.pallas_call(
        _head_body,
        out_shape=jax.ShapeDtypeStruct((1, 1), f32),
    )(h, Wf1, bf1.reshape(1, D // 2), Wf2.reshape(1, D // 2), bf2.reshape(1, 1))
    return out
